# Initial kernel scaffold; baseline (speedup 1.0000x reference)
import jax
import jax.numpy as jnp
from jax.experimental import pallas as pl


def kernel(vectors, codebook):
    raise NotImplementedError("write your pallas kernel here")


# speedup vs baseline: 4.5581x; 4.5581x over previous
import jax
import jax.numpy as jnp
from jax.experimental import pallas as pl

N_CODES = 1024
CODE_DIM = 64
ROWS = 16384
BLK = 1024


def _vq_block(x_ref, cb_ref, cbt_ref, en_ref, o_ref):
    x = x_ref[...]
    xn = jnp.sum(x ** 2, axis=1, keepdims=True)
    mm = jnp.dot(x, cbt_ref[...])
    d = xn + en_ref[...] - 2.0 * mm
    m = jnp.min(d, axis=1, keepdims=True)
    k_iota = jax.lax.broadcasted_iota(jnp.int32, d.shape, 1)
    idx = jnp.min(jnp.where(d == m, k_iota, N_CODES), axis=1, keepdims=True)
    oh = (idx == k_iota).astype(jnp.float32)
    o_ref[...] = jnp.dot(oh, cb_ref[...])


def kernel(vectors, codebook):
    inputs = jnp.transpose(vectors, (0, 2, 3, 1))
    flat = inputs.reshape(-1, CODE_DIM)
    cbt = codebook.T
    en = jnp.sum(codebook ** 2, axis=1)[None, :]
    out = pl.pallas_call(
        _vq_block,
        grid=(ROWS // BLK,),
        in_specs=[
            pl.BlockSpec((BLK, CODE_DIM), lambda i: (i, 0)),
            pl.BlockSpec((N_CODES, CODE_DIM), lambda i: (0, 0)),
            pl.BlockSpec((CODE_DIM, N_CODES), lambda i: (0, 0)),
            pl.BlockSpec((1, N_CODES), lambda i: (0, 0)),
        ],
        out_specs=pl.BlockSpec((BLK, CODE_DIM), lambda i: (i, 0)),
        out_shape=jax.ShapeDtypeStruct((ROWS, CODE_DIM), jnp.float32),
    )(flat, codebook, cbt, en)
    return out.reshape(inputs.shape)
